# baseline (speedup 1.0000x reference)
import jax
import jax.numpy as jnp
from jax import lax
from jax.experimental import pallas as pl
from jax.experimental.pallas import tpu as pltpu
from jax.experimental.pallas import tpu_sc as plsc

_N = 10000
_G = 64
_NT = 16
_FULL = 640
_LAST = _N - 15 * _FULL
_CH = 80


def _fetch(x_hbm, b_hbm, colbuf, bbuf2d, idx2d, sem, lane, base, n_rows):
    n_chunks = n_rows // _CH
    for c in range(n_chunks):
        pltpu.sync_copy(b_hbm.at[pl.ds(base + c * _CH, _CH)], bbuf2d.at[c])
    for c in range(n_chunks):
        for j in range(_CH // 16):
            idx2d[c, pl.ds(j * 16, 16)] = (base + c * _CH + j * 16 + lane) * 128
    copies = [
        pltpu.async_copy(x_hbm.at[idx2d.at[c]],
                         colbuf.at[pl.ds(c * _CH, _CH)], sem)
        for c in range(n_chunks)
    ]
    for d in copies:
        d.wait()


def _scatter(colbuf, bbuf2d, ones, sh_s, sh_c, n_rows):
    for c in range(n_rows // _CH):
        pltpu.sync_copy(colbuf.at[pl.ds(c * _CH, _CH)],
                        sh_s.at[bbuf2d.at[c]], add=True)
        pltpu.sync_copy(ones, sh_c.at[bbuf2d.at[c]], add=True)


def _body(x_hbm, b_hbm, out_hbm, colbuf, bbuf2d, idx2d, ones, sem,
          sh_s, sh_c, t_s, t_c, obuf):
    wid = lax.axis_index("s")
    lane = lax.iota(jnp.int32, 16)
    zeros16 = jnp.zeros((16,), jnp.float32)

    @pl.when(wid == 0)
    def _():
        for j in range(_G // 16):
            obuf[pl.ds(j * 16, 16)] = zeros16
        pltpu.sync_copy(obuf, sh_s)
        pltpu.sync_copy(obuf, sh_c)

    for j in range(_CH // 16):
        ones[pl.ds(j * 16, 16)] = jnp.full((16,), 1.0, jnp.float32)

    @pl.when(wid < _NT - 1)
    def _():
        _fetch(x_hbm, b_hbm, colbuf, bbuf2d, idx2d, sem, lane,
               wid * _FULL, _FULL)

    @pl.when(wid == _NT - 1)
    def _():
        _fetch(x_hbm, b_hbm, colbuf, bbuf2d, idx2d, sem, lane,
               (_NT - 1) * _FULL, _LAST)

    plsc.subcore_barrier()

    @pl.when(wid < _NT - 1)
    def _():
        _scatter(colbuf, bbuf2d, ones, sh_s, sh_c, _FULL)

    @pl.when(wid == _NT - 1)
    def _():
        _scatter(colbuf, bbuf2d, ones, sh_s, sh_c, _LAST)

    plsc.subcore_barrier()

    @pl.when(wid == 0)
    def _():
        pltpu.sync_copy(sh_s, t_s)
        pltpu.sync_copy(sh_c, t_c)
        for j in range(_G // 16):
            obuf[pl.ds(j * 16, 16)] = (t_s[pl.ds(j * 16, 16)]
                                       / t_c[pl.ds(j * 16, 16)])
        pltpu.sync_copy(obuf, out_hbm)


@jax.jit
def _seg_mean(x, batch):
    mesh = plsc.VectorSubcoreMesh(
        core_axis_name="c", subcore_axis_name="s", num_cores=1)
    f = pl.kernel(
        _body,
        out_type=jax.ShapeDtypeStruct((_G,), jnp.float32),
        mesh=mesh,
        compiler_params=pltpu.CompilerParams(needs_layout_passes=False),
        scratch_types=[
            pltpu.VMEM((_FULL,), jnp.float32),
            pltpu.VMEM((_FULL // _CH, _CH), jnp.int32),
            pltpu.VMEM((_FULL // _CH, _CH), jnp.int32),
            pltpu.VMEM((_CH,), jnp.float32),
            pltpu.SemaphoreType.DMA,
            pltpu.VMEM_SHARED((_G,), jnp.float32),
            pltpu.VMEM_SHARED((_G,), jnp.float32),
            pltpu.VMEM((_G,), jnp.float32),
            pltpu.VMEM((_G,), jnp.float32),
            pltpu.VMEM((_G,), jnp.float32),
        ],
    )
    return f(x.reshape(-1), batch)


def kernel(x, edge_index, edge_attr, batch):
    out = _seg_mean(x, batch.astype(jnp.int32))
    return out[:, None]
